# speedup vs baseline: 1.2616x; 1.2616x over previous
import jax
import jax.numpy as jnp
from jax import lax
from jax.experimental import pallas as pl
from jax.experimental.pallas import tpu as pltpu
from jax.experimental.pallas import tpu_sc as plsc

B_ROWS = 500000
D = 128
OUT_D = 4 * D
C = 400
SUB = 80
NC = 2
NS = 16
NW = NC * NS
N_CHUNKS = B_ROWS // C
MAX_J = (N_CHUNKS + NW - 1) // NW
LANES = 16


def _body(z_src, z_dst, idx, t_enc, table, out, idx_v, pos_v, tmp_v, sem):
    wid = lax.axis_index("s") * NC + lax.axis_index("c")

    def chunk_body(j, carry):
        cid = wid + j * NW

        @pl.when(cid < N_CHUNKS)
        def _():
            base = cid * C
            rows = pl.ds(base, C)
            pltpu.sync_copy(idx.at[rows], idx_v)
            pltpu.sync_copy(z_src.at[rows], pos_v)
            pltpu.sync_copy(pos_v, out.at[rows, pl.ds(0, D)])
            copies = []
            for k in range(C // SUB):
                copies.append(
                    pltpu.async_copy(
                        table.at[idx_v.at[pl.ds(k * SUB, SUB)]],
                        tmp_v.at[pl.ds(k * SUB, SUB), :],
                        sem,
                    )
                )
            for cp in copies:
                cp.wait()

            def row_body(r, c2):
                for kk in range(D // LANES):
                    sl = pl.ds(kk * LANES, LANES)
                    pos_v[r, sl] = pos_v[r, sl] + tmp_v[r, sl]
                return c2

            lax.fori_loop(0, C, row_body, 0, unroll=2)
            pltpu.sync_copy(pos_v, out.at[rows, pl.ds(2 * D, D)])

            pltpu.sync_copy(z_dst.at[rows], tmp_v)
            pltpu.sync_copy(tmp_v, out.at[rows, pl.ds(D, D)])
            pltpu.sync_copy(t_enc.at[rows], tmp_v)
            pltpu.sync_copy(tmp_v, out.at[rows, pl.ds(3 * D, D)])

        return carry

    lax.fori_loop(0, MAX_J, chunk_body, 0)


def kernel(z_src, z_dst, raw_msg, t_enc, emb_table):
    mesh = plsc.VectorSubcoreMesh(core_axis_name="c", subcore_axis_name="s")
    run = pl.kernel(
        _body,
        out_type=jax.ShapeDtypeStruct((B_ROWS, OUT_D), jnp.float32),
        mesh=mesh,
        scratch_types=[
            pltpu.VMEM((C,), jnp.int32),
            pltpu.VMEM((C, D), jnp.float32),
            pltpu.VMEM((C, D), jnp.float32),
            pltpu.SemaphoreType.DMA,
        ],
    )
    return run(z_src, z_dst, raw_msg.astype(jnp.int32), t_enc, emb_table)


# speedup vs baseline: 2.2374x; 1.7735x over previous
import jax
import jax.numpy as jnp
from jax import lax
from jax.experimental import pallas as pl
from jax.experimental.pallas import tpu as pltpu
from jax.experimental.pallas import tpu_sc as plsc

B_ROWS = 500000
D = 128
OUT_D = 4 * D
C = 80
NC = 2
NS = 16
NW = NC * NS
N_CHUNKS = B_ROWS // C
MAX_J = (N_CHUNKS + NW - 1) // NW
LANES = 16


def _body(z_src, z_dst, idx, t_enc, table, out,
          idx_v0, idx_v1, zs_v0, zs_v1, zd_v0, zd_v1, te_v0, te_v1,
          g_v0, g_v1,
          sem_i0, sem_i1, sem_l0, sem_l1, sem_g0, sem_g1, sem_w0, sem_w1):
    wid = lax.axis_index("s") * NC + lax.axis_index("c")
    idx_v = (idx_v0, idx_v1)
    zs_v = (zs_v0, zs_v1)
    zd_v = (zd_v0, zd_v1)
    te_v = (te_v0, te_v1)
    g_v = (g_v0, g_v1)
    sem_i = (sem_i0, sem_i1)
    sem_l = (sem_l0, sem_l1)
    sem_g = (sem_g0, sem_g1)
    sem_w = (sem_w0, sem_w1)

    def load_descs(s, rows):
        return (
            pltpu.make_async_copy(z_src.at[rows], zs_v[s], sem_l[s]),
            pltpu.make_async_copy(z_dst.at[rows], zd_v[s], sem_l[s]),
            pltpu.make_async_copy(t_enc.at[rows], te_v[s], sem_l[s]),
        )

    def write_descs(s, rows):
        return (
            pltpu.make_async_copy(zs_v[s], out.at[rows, pl.ds(0, D)], sem_w[s]),
            pltpu.make_async_copy(zd_v[s], out.at[rows, pl.ds(D, D)], sem_w[s]),
            pltpu.make_async_copy(te_v[s], out.at[rows, pl.ds(3 * D, D)], sem_w[s]),
            pltpu.make_async_copy(g_v[s], out.at[rows, pl.ds(2 * D, D)], sem_w[s]),
        )

    def gather_desc(s):
        return pltpu.make_async_copy(table.at[idx_v[s]], g_v[s], sem_g[s])

    def prefetch(s, cid, first):
        rows = pl.ds(cid * C, C)
        if not first:
            for d in write_descs(s, rows):
                d.wait()
        pltpu.make_async_copy(idx.at[rows], idx_v[s], sem_i[s]).start()
        for d in load_descs(s, rows):
            d.start()
        pltpu.make_async_copy(idx.at[rows], idx_v[s], sem_i[s]).wait()
        gather_desc(s).start()

    def process(s, cid):
        rows = pl.ds(cid * C, C)
        for d in load_descs(s, rows):
            d.wait()
        w = write_descs(s, rows)
        w[0].start()
        w[1].start()
        w[2].start()
        gather_desc(s).wait()

        def row_body(r, c2):
            for kk in range(D // LANES):
                sl = pl.ds(kk * LANES, LANES)
                g_v[s][r, sl] = g_v[s][r, sl] + zs_v[s][r, sl]
            return c2

        lax.fori_loop(0, C, row_body, 0, unroll=4)
        w[3].start()

    for s in (0, 1):
        prefetch(s, wid + s * NW, first=True)

    def pair_body(p, carry):
        for s in (0, 1):
            cid = wid + (2 * p + s) * NW
            nxt = cid + 2 * NW

            @pl.when(cid < N_CHUNKS)
            def _():
                process(s, cid)

            @pl.when(nxt < N_CHUNKS)
            def _():
                prefetch(s, nxt, first=False)

        return carry

    lax.fori_loop(0, MAX_J // 2, pair_body, 0)

    for s in (0, 1):
        rows = pl.ds(wid * C, C)
        for d in write_descs(s, rows):
            d.wait()


def kernel(z_src, z_dst, raw_msg, t_enc, emb_table):
    mesh = plsc.VectorSubcoreMesh(core_axis_name="c", subcore_axis_name="s")
    run = pl.kernel(
        _body,
        out_type=jax.ShapeDtypeStruct((B_ROWS, OUT_D), jnp.float32),
        mesh=mesh,
        scratch_types=(
            [pltpu.VMEM((C,), jnp.int32)] * 2
            + [pltpu.VMEM((C, D), jnp.float32)] * 8
            + [pltpu.SemaphoreType.DMA] * 8
        ),
    )
    return run(z_src, z_dst, raw_msg.astype(jnp.int32), t_enc, emb_table)


# speedup vs baseline: 2.4239x; 1.0833x over previous
import jax
import jax.numpy as jnp
from jax import lax
from jax.experimental import pallas as pl
from jax.experimental.pallas import tpu as pltpu
from jax.experimental.pallas import tpu_sc as plsc

B_ROWS = 500000
D = 128
OUT_D = 4 * D
C = 80
NC = 2
NS = 16
NW = NC * NS
N_CHUNKS = B_ROWS // C
MAX_J = (N_CHUNKS + NW - 1) // NW
N_TRIPLES = (MAX_J + 2) // 3
NBUF = 3
LANES = 16


def _body(z_src, z_dst, idx, t_enc, table, out,
          idx_v0, idx_v1, idx_v2,
          zs_v0, zs_v1, zs_v2, zd_v0, zd_v1, zd_v2,
          te_v0, te_v1, te_v2, g_v0, g_v1, g_v2,
          sem_i0, sem_i1, sem_i2, sem_l0, sem_l1, sem_l2,
          sem_g0, sem_g1, sem_g2, sem_w0, sem_w1, sem_w2):
    wid = lax.axis_index("s") * NC + lax.axis_index("c")
    idx_v = (idx_v0, idx_v1, idx_v2)
    zs_v = (zs_v0, zs_v1, zs_v2)
    zd_v = (zd_v0, zd_v1, zd_v2)
    te_v = (te_v0, te_v1, te_v2)
    g_v = (g_v0, g_v1, g_v2)
    sem_i = (sem_i0, sem_i1, sem_i2)
    sem_l = (sem_l0, sem_l1, sem_l2)
    sem_g = (sem_g0, sem_g1, sem_g2)
    sem_w = (sem_w0, sem_w1, sem_w2)

    def load_descs(s, rows):
        return (
            pltpu.make_async_copy(z_src.at[rows], zs_v[s], sem_l[s]),
            pltpu.make_async_copy(z_dst.at[rows], zd_v[s], sem_l[s]),
            pltpu.make_async_copy(t_enc.at[rows], te_v[s], sem_l[s]),
        )

    def write_descs(s, rows):
        return (
            pltpu.make_async_copy(zs_v[s], out.at[rows, pl.ds(0, D)], sem_w[s]),
            pltpu.make_async_copy(zd_v[s], out.at[rows, pl.ds(D, D)], sem_w[s]),
            pltpu.make_async_copy(te_v[s], out.at[rows, pl.ds(3 * D, D)], sem_w[s]),
            pltpu.make_async_copy(g_v[s], out.at[rows, pl.ds(2 * D, D)], sem_w[s]),
        )

    def issue_loads(s, cid):
        rows = pl.ds(cid * C, C)
        pltpu.make_async_copy(idx.at[rows], idx_v[s], sem_i[s]).start()
        for d in load_descs(s, rows):
            d.start()

    def rows_of(cid):
        return pl.ds(cid * C, C)

    issue_loads(0, wid)
    issue_loads(1, wid + NW)
    pltpu.make_async_copy(idx.at[rows_of(wid)], idx_v[0], sem_i[0]).wait()
    pltpu.make_async_copy(table.at[idx_v[0]], g_v[0], sem_g[0]).start()

    def triple_body(t, carry):
        for u in range(NBUF):
            jv = NBUF * t + u
            cid = wid + jv * NW
            s = u
            s1 = (u + 1) % NBUF
            s2 = (u + 2) % NBUF

            @pl.when(cid < N_CHUNKS)
            def _():
                rows = rows_of(cid)
                for d in load_descs(s, rows):
                    d.wait()
                w = write_descs(s, rows)
                w[0].start()
                w[1].start()
                w[2].start()

            @pl.when(cid + NW < N_CHUNKS)
            def _():
                rows1 = rows_of(cid + NW)
                pltpu.make_async_copy(idx.at[rows1], idx_v[s1], sem_i[s1]).wait()
                pltpu.make_async_copy(table.at[idx_v[s1]], g_v[s1], sem_g[s1]).start()

            @pl.when(cid < N_CHUNKS)
            def _():
                rows = rows_of(cid)
                pltpu.make_async_copy(table.at[idx_v[s]], g_v[s], sem_g[s]).wait()

                def row_body(r, c2):
                    for kk in range(D // LANES):
                        sl = pl.ds(kk * LANES, LANES)
                        g_v[s][r, sl] = g_v[s][r, sl] + zs_v[s][r, sl]
                    return c2

                lax.fori_loop(0, C, row_body, 0, unroll=4)
                write_descs(s, rows)[3].start()

            has_prev = (cid + 2 * NW < N_CHUNKS)
            if u == 0:
                has_prev = has_prev & (t >= 1)

            @pl.when(has_prev)
            def _():
                for d in write_descs(s2, rows_of(cid - NW)):
                    d.wait()

            @pl.when(cid + 2 * NW < N_CHUNKS)
            def _():
                issue_loads(s2, cid + 2 * NW)

        return carry

    lax.fori_loop(0, N_TRIPLES, triple_body, 0)

    for s in range(NBUF):
        for d in write_descs(s, rows_of(wid)):
            d.wait()


def kernel(z_src, z_dst, raw_msg, t_enc, emb_table):
    mesh = plsc.VectorSubcoreMesh(core_axis_name="c", subcore_axis_name="s")
    run = pl.kernel(
        _body,
        out_type=jax.ShapeDtypeStruct((B_ROWS, OUT_D), jnp.float32),
        mesh=mesh,
        scratch_types=(
            [pltpu.VMEM((C,), jnp.int32)] * 3
            + [pltpu.VMEM((C, D), jnp.float32)] * 12
            + [pltpu.SemaphoreType.DMA] * 12
        ),
    )
    return run(z_src, z_dst, raw_msg.astype(jnp.int32), t_enc, emb_table)


# speedup vs baseline: 2.4969x; 1.0301x over previous
import jax
import jax.numpy as jnp
from jax import lax
from jax.experimental import pallas as pl
from jax.experimental.pallas import tpu as pltpu
from jax.experimental.pallas import tpu_sc as plsc

B_ROWS = 500000
D = 128
OUT_D = 4 * D
C = 80
NC = 2
NS = 16
NW = NC * NS
N_CHUNKS = B_ROWS // C
MAX_J = (N_CHUNKS + NW - 1) // NW
N_TRIPLES = (MAX_J + 2) // 3
NBUF = 3
LANES = 16


def _body(z_src, z_dst, idx, t_enc, table, out,
          idx_v0, idx_v1, idx_v2,
          zs_v0, zs_v1, zs_v2, zd_v0, zd_v1, zd_v2,
          te_v0, te_v1, te_v2, g_v0, g_v1, g_v2,
          sem_i0, sem_i1, sem_i2, sem_l0, sem_l1, sem_l2,
          sem_g0, sem_g1, sem_g2, sem_w0, sem_w1, sem_w2):
    wid = lax.axis_index("s") * NC + lax.axis_index("c")
    idx_v = (idx_v0, idx_v1, idx_v2)
    zs_v = (zs_v0, zs_v1, zs_v2)
    zd_v = (zd_v0, zd_v1, zd_v2)
    te_v = (te_v0, te_v1, te_v2)
    g_v = (g_v0, g_v1, g_v2)
    sem_i = (sem_i0, sem_i1, sem_i2)
    sem_l = (sem_l0, sem_l1, sem_l2)
    sem_g = (sem_g0, sem_g1, sem_g2)
    sem_w = (sem_w0, sem_w1, sem_w2)

    def load_descs(s, rows):
        return (
            pltpu.make_async_copy(z_src.at[rows], zs_v[s], sem_l[s]),
            pltpu.make_async_copy(z_dst.at[rows], zd_v[s], sem_l[s]),
            pltpu.make_async_copy(t_enc.at[rows], te_v[s], sem_l[s]),
        )

    def write_descs(s, rows):
        return (
            pltpu.make_async_copy(zs_v[s], out.at[rows, pl.ds(0, D)], sem_w[s]),
            pltpu.make_async_copy(zd_v[s], out.at[rows, pl.ds(D, D)], sem_w[s]),
            pltpu.make_async_copy(te_v[s], out.at[rows, pl.ds(3 * D, D)], sem_w[s]),
            pltpu.make_async_copy(g_v[s], out.at[rows, pl.ds(2 * D, D)], sem_w[s]),
        )

    def issue_loads(s, cid):
        rows = pl.ds(cid * C, C)
        pltpu.make_async_copy(idx.at[rows], idx_v[s], sem_i[s]).start()
        for d in load_descs(s, rows):
            d.start()

    def rows_of(cid):
        return pl.ds(cid * C, C)

    issue_loads(0, wid)
    issue_loads(1, wid + NW)
    pltpu.make_async_copy(idx.at[rows_of(wid)], idx_v[0], sem_i[0]).wait()
    pltpu.make_async_copy(table.at[idx_v[0]], g_v[0], sem_g[0]).start()

    def triple_body(t, carry):
        for u in range(NBUF):
            jv = NBUF * t + u
            cid = wid + jv * NW
            s = u
            s1 = (u + 1) % NBUF
            s2 = (u + 2) % NBUF

            @pl.when(cid < N_CHUNKS)
            def _():
                rows = rows_of(cid)
                for d in load_descs(s, rows):
                    d.wait()
                w = write_descs(s, rows)
                w[0].start()
                w[1].start()
                w[2].start()

            @pl.when(cid + NW < N_CHUNKS)
            def _():
                rows1 = rows_of(cid + NW)
                pltpu.make_async_copy(idx.at[rows1], idx_v[s1], sem_i[s1]).wait()
                pltpu.make_async_copy(table.at[idx_v[s1]], g_v[s1], sem_g[s1]).start()

            @pl.when(cid < N_CHUNKS)
            def _():
                rows = rows_of(cid)
                pltpu.make_async_copy(table.at[idx_v[s]], g_v[s], sem_g[s]).wait()

                def row_body(r, c2):
                    for kk in range(D // LANES):
                        sl = pl.ds(kk * LANES, LANES)
                        plsc.addupdate(g_v[s].at[r, sl], zs_v[s][r, sl])
                    return c2

                lax.fori_loop(0, C, row_body, 0, unroll=4)
                write_descs(s, rows)[3].start()

            has_prev = (cid + 2 * NW < N_CHUNKS)
            if u == 0:
                has_prev = has_prev & (t >= 1)

            @pl.when(has_prev)
            def _():
                for d in write_descs(s2, rows_of(cid - NW)):
                    d.wait()

            @pl.when(cid + 2 * NW < N_CHUNKS)
            def _():
                issue_loads(s2, cid + 2 * NW)

        return carry

    lax.fori_loop(0, N_TRIPLES, triple_body, 0)

    for s in range(NBUF):
        for d in write_descs(s, rows_of(wid)):
            d.wait()


def kernel(z_src, z_dst, raw_msg, t_enc, emb_table):
    mesh = plsc.VectorSubcoreMesh(core_axis_name="c", subcore_axis_name="s")
    run = pl.kernel(
        _body,
        out_type=jax.ShapeDtypeStruct((B_ROWS, OUT_D), jnp.float32),
        mesh=mesh,
        scratch_types=(
            [pltpu.VMEM((C,), jnp.int32)] * 3
            + [pltpu.VMEM((C, D), jnp.float32)] * 12
            + [pltpu.SemaphoreType.DMA] * 12
        ),
    )
    return run(z_src, z_dst, raw_msg.astype(jnp.int32), t_enc, emb_table)
